# baseline (speedup 1.0000x reference)
import jax
import jax.numpy as jnp
from jax import lax
from jax.experimental import pallas as pl
from jax.experimental.pallas import tpu as pltpu
from jax.experimental.pallas import tpu_sc as plsc

NC = 2
NS = 16
NW = NC * NS

B = 4096 * 200
D = 256
B_PER_W = B // NW
CHUNK = 80
N_CHUNKS = B_PER_W // CHUNK
NBUF = 4
F = 2


def _gather_body(idx_hbm, table_hbm, out_hbm, idx_v,
                 r0, r1, r2, r3, g0, g1, g2, g3, o0, o1, o2, o3):
    rows = (r0, r1, r2, r3)
    gsem = (g0, g1, g2, g3)
    osem = (o0, o1, o2, o3)
    wid = lax.axis_index("s") * NC + lax.axis_index("c")
    base = wid * B_PER_W
    pltpu.sync_copy(idx_hbm.at[wid], idx_v)

    for k in range(F):
        pltpu.async_copy(table_hbm.at[idx_v.at[k]], rows[k], gsem[k])

    def ring(kk, carry):
        for b in range(NBUF):
            k = kk * NBUF + b
            kf = k + F
            sf = (b + F) % NBUF

            @pl.when(kf >= NBUF)
            def _free_slot():
                pltpu.make_async_copy(
                    rows[sf],
                    out_hbm.at[pl.ds(base + (kf - NBUF) * CHUNK, CHUNK)],
                    osem[sf],
                ).wait()

            pltpu.async_copy(table_hbm.at[idx_v.at[kf]], rows[sf], gsem[sf])
            pltpu.make_async_copy(
                table_hbm.at[idx_v.at[k]], rows[b], gsem[b]
            ).wait()
            pltpu.async_copy(
                rows[b], out_hbm.at[pl.ds(base + k * CHUNK, CHUNK)], osem[b]
            )
        return carry

    lax.fori_loop(0, N_CHUNKS // NBUF - 1, ring, 0)

    for b in range(NBUF):
        k = N_CHUNKS - NBUF + b
        kf = k + F
        if kf < N_CHUNKS:
            sf = kf % NBUF
            pltpu.make_async_copy(
                rows[sf],
                out_hbm.at[pl.ds(base + (kf - NBUF) * CHUNK, CHUNK)],
                osem[sf],
            ).wait()
            pltpu.async_copy(table_hbm.at[idx_v.at[kf]], rows[sf], gsem[sf])
        pltpu.make_async_copy(
            table_hbm.at[idx_v.at[k]], rows[b], gsem[b]
        ).wait()
        pltpu.async_copy(
            rows[b], out_hbm.at[pl.ds(base + k * CHUNK, CHUNK)], osem[b]
        )
    for b in range(NBUF):
        k = N_CHUNKS - NBUF + b
        pltpu.make_async_copy(
            rows[b], out_hbm.at[pl.ds(base + k * CHUNK, CHUNK)], osem[b]
        ).wait()


def kernel(frameIndices, numFrames, frameEmbed_weight):
    del numFrames
    idx = frameIndices.astype(jnp.int32).reshape(NW, N_CHUNKS, CHUNK)
    mesh = plsc.VectorSubcoreMesh(
        core_axis_name="c", subcore_axis_name="s", num_cores=NC, num_subcores=NS
    )
    out = pl.kernel(
        _gather_body,
        out_type=jax.ShapeDtypeStruct((B, D), jnp.float32),
        mesh=mesh,
        scratch_types=(
            [pltpu.VMEM((N_CHUNKS, CHUNK), jnp.int32)]
            + [pltpu.VMEM((CHUNK, D), jnp.float32) for _ in range(NBUF)]
            + [pltpu.SemaphoreType.DMA for _ in range(2 * NBUF)]
        ),
    )(idx, frameEmbed_weight)
    return out.reshape(frameIndices.shape[0], frameIndices.shape[1], D)
